# baseline (speedup 1.0000x reference)
import functools

import jax
import jax.numpy as jnp
from jax import lax
from jax.experimental import pallas as pl
from jax.experimental.pallas import tpu as pltpu
from jax.experimental.pallas import tpu_sc as plsc

_N = 10000
_E = 320000
_D = 128
_G = 64
_H = 64
_WIDTH = 4
_NCLASS = 2

_NC = 2
_NS = 16
_NW = _NC * _NS
_EPW = _E // _NW
_CH = 80
_NCHUNK = 128
_EPAD = _NW * _NCHUNK * _CH
_NCH0 = 184
_NCH1 = 72
_SRCROWS = _NS * _NCH0 + _NS * _NCH1 + (_NCH0 - _NCH1)
_NPAD = 10240
_RPW = _NPAD // _NS
_DROWS = _NPAD // 16
_DRPW = _DROWS // _NS
_DCH = 2000
_NIB = _DROWS // 128

_R = 2048
_NBLK = _NPAD // _R


def _act(z):
    return jnp.where(z >= 0, z, 0.01 * z)


@functools.cache
def _sc_scatter_fn():
    mesh = plsc.VectorSubcoreMesh(core_axis_name="c", subcore_axis_name="s")

    @functools.partial(
        pl.kernel,
        out_type=jax.ShapeDtypeStruct((_NC, _NPAD, _D), jnp.float32),
        mesh=mesh,
        compiler_params=pltpu.CompilerParams(needs_layout_passes=False),
        scratch_types=[
            pltpu.VMEM((_NCH0, _CH), jnp.int32),
            pltpu.VMEM((_CH,), jnp.int32),
            pltpu.VMEM((_CH,), jnp.int32),
            pltpu.VMEM((_CH, _D), jnp.float32),
            pltpu.VMEM((_CH, _D), jnp.float32),
            pltpu.VMEM_SHARED((_NPAD, _D), jnp.float32),
            pltpu.SemaphoreType.DMA,
            pltpu.SemaphoreType.DMA,
            pltpu.SemaphoreType.DMA,
            pltpu.SemaphoreType.DMA,
        ],
    )
    def scatter_rows(table, src2, src1, dst1, zrows, out,
                     srcbuf, dsti0, dsti1, rows0, rows1, acc,
                     semr0, semr1, semd0, semd1):
        c = lax.axis_index("c")
        s = lax.axis_index("s")
        cb = pl.multiple_of(
            jnp.where(c == 0, s * _NCH0, _NS * _NCH0 + s * _NCH1), 8)
        nch = jnp.where(c == 0, _NCH0, _NCH1)
        ebase = pl.multiple_of(cb * _CH, 8)
        pltpu.sync_copy(src2.at[pl.ds(cb, _NCH0)], srcbuf)
        pltpu.sync_copy(zrows.at[pl.ds(s * _RPW, _RPW)],
                        acc.at[pl.ds(s * _RPW, _RPW)])
        plsc.subcore_barrier()

        @pl.when(c == 0)
        def _():
            pltpu.async_copy(table.at[srcbuf.at[0]], rows0, semr0)
            pltpu.async_copy(dst1.at[pl.ds(ebase, _CH)], dsti0, semd0)
            pltpu.async_copy(table.at[srcbuf.at[1]], rows1, semr1)
            pltpu.async_copy(dst1.at[pl.ds(ebase + _CH, _CH)], dsti1, semd1)

            def body(i, carry):
                j0 = 2 * i
                pltpu.make_async_copy(table.at[srcbuf.at[0]], rows0, semr0).wait()
                pltpu.make_async_copy(dst1.at[pl.ds(0, _CH)], dsti0, semd0).wait()
                pltpu.sync_copy(rows0, acc.at[dsti0], add=True)

                @pl.when(j0 + 2 < _NCH0)
                def _():
                    pltpu.async_copy(table.at[srcbuf.at[j0 + 2]], rows0, semr0)
                    pltpu.async_copy(
                        dst1.at[pl.ds(ebase + (j0 + 2) * _CH, _CH)], dsti0, semd0)

                pltpu.make_async_copy(table.at[srcbuf.at[1]], rows1, semr1).wait()
                pltpu.make_async_copy(dst1.at[pl.ds(0, _CH)], dsti1, semd1).wait()
                pltpu.sync_copy(rows1, acc.at[dsti1], add=True)

                @pl.when(j0 + 3 < _NCH0)
                def _():
                    pltpu.async_copy(table.at[srcbuf.at[j0 + 3]], rows1, semr1)
                    pltpu.async_copy(
                        dst1.at[pl.ds(ebase + (j0 + 3) * _CH, _CH)], dsti1, semd1)

                return carry

            lax.fori_loop(0, _NCH0 // 2, body, 0)

        @pl.when(c == 1)
        def _():
            def sbody(j, carry):
                pltpu.sync_copy(src1.at[pl.ds(ebase + j * _CH, _CH)], dsti1)
                pltpu.sync_copy(dst1.at[pl.ds(ebase + j * _CH, _CH)], dsti0)
                pltpu.async_copy(table.at[dsti1], rows0, semr0).wait()
                pltpu.sync_copy(rows0, acc.at[dsti0], add=True)
                return carry

            lax.fori_loop(0, _NCH1, sbody, 0)

        plsc.subcore_barrier()
        pltpu.sync_copy(acc.at[pl.ds(s * _RPW, _RPW)],
                        out.at[c, pl.ds(s * _RPW, _RPW)])

    return scatter_rows


@functools.cache
def _sc_deg_fn():
    mesh = plsc.VectorSubcoreMesh(core_axis_name="c", subcore_axis_name="s")

    @functools.partial(
        pl.kernel,
        out_type=jax.ShapeDtypeStruct((_NC, _NPAD), jnp.float32),
        mesh=mesh,
        compiler_params=pltpu.CompilerParams(needs_layout_passes=False),
        scratch_types=[
            pltpu.VMEM((_NPAD,), jnp.float32),
            pltpu.VMEM((_DCH,), jnp.int32),
            pltpu.VMEM((_NS, _RPW), jnp.float32),
            pltpu.VMEM((_RPW,), jnp.float32),
            pltpu.VMEM_SHARED((_NS, _NPAD), jnp.float32),
        ],
    )
    def deg_kernel(dst, zdeg, out, hist, idxbuf, buf, res, stage):
        c = lax.axis_index("c")
        s = lax.axis_index("s")
        wid = c * _NS + s
        pltpu.sync_copy(zdeg, hist)
        ones16 = jnp.full((16,), 1.0, jnp.float32)
        for jc in range(_EPW // _DCH):
            base = pl.multiple_of(wid * _EPW + jc * _DCH, 8)
            pltpu.sync_copy(dst.at[pl.ds(base, _DCH)], idxbuf)

            def body(k, carry):
                idx16 = idxbuf[pl.ds(k * 16, 16)]
                plsc.addupdate_scatter(hist, [idx16], ones16)
                return carry

            lax.fori_loop(0, _DCH // 16, body, 0)
        pltpu.sync_copy(hist, stage.at[s])
        plsc.subcore_barrier()
        pltpu.sync_copy(stage.at[:, pl.ds(s * _RPW, _RPW)], buf)

        def rbody(j, carry):
            v = buf[0, pl.ds(j * 16, 16)]
            for t in range(1, _NS):
                v = v + buf[t, pl.ds(j * 16, 16)]
            res[pl.ds(j * 16, 16)] = v
            return carry

        lax.fori_loop(0, _RPW // 16, rbody, 0)
        pltpu.sync_copy(res, out.at[c, pl.ds(s * _RPW, _RPW)])

    return deg_kernel


def _tc_pre(x, dega, degb, W1, b1, W2, b2):
    def body(x_ref, da, db, w1, b1r, w2, b2r, h_ref, hp_ref, dinv_ref):
        deg = da[...] + db[...]
        good = deg > 0
        dinv = jnp.where(good, lax.rsqrt(jnp.where(good, deg, 1.0)), 0.0)
        z = _act(jnp.dot(x_ref[...], w1[...],
                         preferred_element_type=jnp.float32) + b1r[...])
        h = _act(jnp.dot(z, w2[...],
                         preferred_element_type=jnp.float32) + b2r[...])
        h_ref[...] = h
        hp_ref[...] = h * dinv
        dinv_ref[...] = dinv

    rowspec = pl.BlockSpec((_R, _D), lambda i: (i, 0))
    colspec = pl.BlockSpec((_R, 1), lambda i: (i, 0))
    wspec = pl.BlockSpec((_D, _D), lambda i: (0, 0))
    bspec = pl.BlockSpec((1, _D), lambda i: (0, 0))
    return pl.pallas_call(
        body,
        grid=(_NBLK,),
        in_specs=[rowspec, colspec, colspec, wspec, bspec, wspec, bspec],
        out_specs=[rowspec, rowspec, colspec],
        out_shape=[
            jax.ShapeDtypeStruct((_NPAD, _D), jnp.float32),
            jax.ShapeDtypeStruct((_NPAD, _D), jnp.float32),
            jax.ShapeDtypeStruct((_NPAD, 1), jnp.float32),
        ],
    )(x, dega, degb, W1, b1.reshape(1, _D), W2, b2.reshape(1, _D))


def _tc_mid(s1a, s1b, dinv):
    def body(a, b, dv_ref, tx1_ref, u_ref):
        dv = dv_ref[...]
        tx1 = -dv * (a[...] + b[...])
        tx1_ref[...] = tx1
        u_ref[...] = dv * tx1

    rowspec = pl.BlockSpec((_R, _D), lambda i: (i, 0))
    colspec = pl.BlockSpec((_R, 1), lambda i: (i, 0))
    return pl.pallas_call(
        body,
        grid=(_NBLK,),
        in_specs=[rowspec, rowspec, colspec],
        out_specs=[rowspec, rowspec],
        out_shape=[
            jax.ShapeDtypeStruct((_NPAD, _D), jnp.float32),
            jax.ShapeDtypeStruct((_NPAD, _D), jnp.float32),
        ],
    )(s1a, s1b, dinv)


def _tc_post(h, tx1, s2a, s2b, dinv, gp, gpT, Wc0, Wc1, Wc2, bc,
             W3, b3, W4, b4, xLx, W8, b8, W9, b9, W5, b5, W6, b6,
             W7a, W7b, b7):
    DW = _WIDTH * _D

    def body(h_ref, tx1_ref, a_ref, b_ref, dv_ref, gp_ref, gpt_ref,
             wc0, wc1, wc2, bcr, w3, b3r, w4, b4r, xlx, w8, b8r, w9, b9r,
             w5, b5r, w6, b6r, w7a, w7b, b7r, out_ref, hg_acc):
        i = pl.program_id(0)

        @pl.when(i == 0)
        def _():
            hg_acc[...] = jnp.zeros_like(hg_acc)

        tx2 = -2.0 * dv_ref[...] * (a_ref[...] + b_ref[...]) - h_ref[...]
        hf = (jnp.dot(h_ref[...], wc0[...], preferred_element_type=jnp.float32)
              + jnp.dot(tx1_ref[...], wc1[...], preferred_element_type=jnp.float32)
              + jnp.dot(tx2, wc2[...], preferred_element_type=jnp.float32)
              + bcr[...])
        h2 = _act(jnp.dot(hf, w3[...], preferred_element_type=jnp.float32) + b3r[...])
        h2 = _act(jnp.dot(h2, w4[...], preferred_element_type=jnp.float32) + b4r[...])
        t = _act(jnp.dot(xlx[...], w8[...], preferred_element_type=jnp.float32) + b8r[...])
        t = _act(jnp.dot(t, w9[...], preferred_element_type=jnp.float32) + b9r[...])
        oh = (gpt_ref[...] > 0).astype(jnp.float32)
        tg = jnp.dot(oh, t, preferred_element_type=jnp.float32)
        scores = jnp.sum(h2 * tg, axis=1, keepdims=True)
        w = h2 * scores
        hg_acc[...] += jnp.dot(gp_ref[...], w, preferred_element_type=jnp.float32)

        @pl.when(i == _NBLK - 1)
        def _():
            xl = jnp.dot(xlx[...], w5[...], preferred_element_type=jnp.float32) + b5r[...]
            xl = _act(jnp.dot(xl, w6[...], preferred_element_type=jnp.float32) + b6r[...])
            out_ref[...] = (
                jnp.dot(hg_acc[...], w7a[...], preferred_element_type=jnp.float32)
                + jnp.dot(xl, w7b[...], preferred_element_type=jnp.float32)
                + b7r[...])

    rowspec = pl.BlockSpec((_R, _D), lambda i: (i, 0))
    colspec = pl.BlockSpec((_R, 1), lambda i: (i, 0))

    def cspec(shape):
        return pl.BlockSpec(shape, lambda i: tuple(0 for _ in shape))

    return pl.pallas_call(
        body,
        grid=(_NBLK,),
        in_specs=[
            rowspec, rowspec, rowspec, rowspec, colspec,
            pl.BlockSpec((_G, _R), lambda i: (0, i)),
            pl.BlockSpec((_R, _G), lambda i: (i, 0)),
            cspec((_D, DW)), cspec((_D, DW)), cspec((_D, DW)), cspec((1, DW)),
            cspec((DW, _H)), cspec((1, _H)), cspec((_H, _H)), cspec((1, _H)),
            cspec((_G, _D)), cspec((_D, _H)), cspec((1, _H)),
            cspec((_H, _H)), cspec((1, _H)),
            cspec((_D, _H)), cspec((1, _H)), cspec((_H, _H)), cspec((1, _H)),
            cspec((_H, _NCLASS)), cspec((_H, _NCLASS)), cspec((1, _NCLASS)),
        ],
        out_specs=pl.BlockSpec((_G, _NCLASS), lambda i: (0, 0)),
        out_shape=jax.ShapeDtypeStruct((_G, _NCLASS), jnp.float32),
        scratch_shapes=[pltpu.VMEM((_G, _H), jnp.float32)],
    )(h, tx1, s2a, s2b, dinv, gp, gpT,
      Wc0, Wc1, Wc2, bc, W3, b3.reshape(1, _H), W4, b4.reshape(1, _H),
      xLx, W8, b8.reshape(1, _H), W9, b9.reshape(1, _H),
      W5, b5.reshape(1, _H), W6, b6.reshape(1, _H),
      W7a, W7b, b7.reshape(1, _NCLASS))


def kernel(features_list, edge_index, xLx_batch, graph_id, graphpool,
           W1, b1, W2, b2, W3, b3, W4, b4, W5, b5, W6, b6, W7, b7,
           W8, b8, W9, b9, cheb_W, cheb_b):
    del graph_id
    src = edge_index[0]
    dst = edge_index[1]
    npad_s = _SRCROWS * _CH - _E
    npad_e = _EPAD - _E
    src2 = jnp.concatenate([src, jnp.full((npad_s,), _N, jnp.int32)]
                           ).reshape(_SRCROWS, _CH)
    padrows = _N + (jnp.arange(npad_e, dtype=jnp.int32) % (_NPAD - _N))
    dst1 = jnp.concatenate([dst, padrows])
    zrows = jnp.zeros((_NPAD, _D), jnp.float32)
    zdeg = jnp.zeros((_NPAD,), jnp.float32)
    xpad = jnp.pad(features_list, ((0, _NPAD - _N), (0, 0)))
    gp = jnp.pad(graphpool, ((0, 0), (0, _NPAD - _N)))

    degs = _sc_deg_fn()(dst, zdeg).reshape(_NC, _NPAD, 1)
    h, hp, dinv = _tc_pre(xpad, degs[0], degs[1], W1, b1, W2, b2)
    src1 = src2.reshape(-1)
    s1 = _sc_scatter_fn()(hp, src2, src1, dst1, zrows)
    tx1, u = _tc_mid(s1[0], s1[1], dinv)
    s2 = _sc_scatter_fn()(u, src2, src1, dst1, zrows)

    Wc0 = jnp.transpose(cheb_W[:, 0], (1, 0, 2)).reshape(_D, _WIDTH * _D)
    Wc1 = jnp.transpose(cheb_W[:, 1], (1, 0, 2)).reshape(_D, _WIDTH * _D)
    Wc2 = jnp.transpose(cheb_W[:, 2], (1, 0, 2)).reshape(_D, _WIDTH * _D)
    bc = cheb_b.reshape(1, _WIDTH * _D)
    gpT = gp.T

    return _tc_post(h, tx1, s2[0], s2[1], dinv, gp, gpT,
                    Wc0, Wc1, Wc2, bc, W3, b3, W4, b4,
                    xLx_batch, W8, b8, W9, b9, W5, b5, W6, b6,
                    W7[:_H], W7[_H:], b7)


# speedup vs baseline: 1.5848x; 1.5848x over previous
import functools

import jax
import jax.numpy as jnp
from jax import lax
from jax.experimental import pallas as pl
from jax.experimental.pallas import tpu as pltpu
from jax.experimental.pallas import tpu_sc as plsc

_N = 10000
_E = 320000
_D = 128
_G = 64
_H = 64
_WIDTH = 4
_NCLASS = 2

_NC = 2
_NS = 16
_NW = _NC * _NS
_EPW = _E // _NW
_CH = 80
_NCHUNK = 128
_EPAD = _NW * _NCHUNK * _CH
_NCH0 = 184
_NCH1 = 72
_SRCROWS = _NS * _NCH0 + _NS * _NCH1 + (_NCH0 - _NCH1)
_NPAD = 10240
_RPW = _NPAD // _NS
_DROWS = _NPAD // 16
_DRPW = _DROWS // _NS
_DCH = 2000
_NIB = _DROWS // 128

_R = 2048
_NBLK = _NPAD // _R


def _act(z):
    return jnp.where(z >= 0, z, 0.01 * z)


@functools.cache
def _sc_scatter_fn():
    mesh = plsc.VectorSubcoreMesh(core_axis_name="c", subcore_axis_name="s")

    @functools.partial(
        pl.kernel,
        out_type=jax.ShapeDtypeStruct((_NC, _NPAD, _D), jnp.float32),
        mesh=mesh,
        compiler_params=pltpu.CompilerParams(needs_layout_passes=False),
        scratch_types=[
            pltpu.VMEM((_NCH0, _CH), jnp.int32),
            pltpu.VMEM((_CH,), jnp.int32),
            pltpu.VMEM((_CH,), jnp.int32),
            pltpu.VMEM((_CH, _D), jnp.float32),
            pltpu.VMEM((_CH, _D), jnp.float32),
            pltpu.VMEM_SHARED((_NPAD, _D), jnp.float32),
            pltpu.SemaphoreType.DMA,
            pltpu.SemaphoreType.DMA,
            pltpu.SemaphoreType.DMA,
            pltpu.SemaphoreType.DMA,
        ],
    )
    def scatter_rows(table, src2, src1, dst1, zrows, out,
                     srcbuf, dsti0, dsti1, rows0, rows1, acc,
                     semr0, semr1, semd0, semd1):
        c = lax.axis_index("c")
        s = lax.axis_index("s")
        cb = pl.multiple_of(
            jnp.where(c == 0, s * _NCH0, _NS * _NCH0 + s * _NCH1), 8)
        nch = jnp.where(c == 0, _NCH0, _NCH1)
        ebase = pl.multiple_of(cb * _CH, 8)
        pltpu.sync_copy(src2.at[pl.ds(cb, _NCH0)], srcbuf)
        pltpu.sync_copy(zrows.at[pl.ds(s * _RPW, _RPW)],
                        acc.at[pl.ds(s * _RPW, _RPW)])
        plsc.subcore_barrier()

        @pl.when(c == 0)
        def _():
            pltpu.async_copy(table.at[srcbuf.at[0]], rows0, semr0)
            pltpu.async_copy(dst1.at[pl.ds(ebase, _CH)], dsti0, semd0)
            pltpu.async_copy(table.at[srcbuf.at[1]], rows1, semr1)
            pltpu.async_copy(dst1.at[pl.ds(ebase + _CH, _CH)], dsti1, semd1)

            def body(i, carry):
                j0 = 2 * i
                pltpu.make_async_copy(table.at[srcbuf.at[0]], rows0, semr0).wait()
                pltpu.make_async_copy(dst1.at[pl.ds(0, _CH)], dsti0, semd0).wait()
                pltpu.sync_copy(rows0, acc.at[dsti0], add=True)

                @pl.when(j0 + 2 < _NCH0)
                def _():
                    pltpu.async_copy(table.at[srcbuf.at[j0 + 2]], rows0, semr0)
                    pltpu.async_copy(
                        dst1.at[pl.ds(ebase + (j0 + 2) * _CH, _CH)], dsti0, semd0)

                pltpu.make_async_copy(table.at[srcbuf.at[1]], rows1, semr1).wait()
                pltpu.make_async_copy(dst1.at[pl.ds(0, _CH)], dsti1, semd1).wait()
                pltpu.sync_copy(rows1, acc.at[dsti1], add=True)

                @pl.when(j0 + 3 < _NCH0)
                def _():
                    pltpu.async_copy(table.at[srcbuf.at[j0 + 3]], rows1, semr1)
                    pltpu.async_copy(
                        dst1.at[pl.ds(ebase + (j0 + 3) * _CH, _CH)], dsti1, semd1)

                return carry

            lax.fori_loop(0, 0, body, 0)

        @pl.when(c == 1)
        def _():
            def sbody(j, carry):
                pltpu.sync_copy(src1.at[pl.ds(ebase + j * _CH, _CH)], dsti1)
                pltpu.sync_copy(dst1.at[pl.ds(ebase + j * _CH, _CH)], dsti0)
                pltpu.async_copy(table.at[dsti1], rows0, semr0).wait()
                pltpu.sync_copy(rows0, acc.at[dsti0], add=True)
                return carry

            lax.fori_loop(0, _NCH1, sbody, 0)

        plsc.subcore_barrier()
        pltpu.sync_copy(acc.at[pl.ds(s * _RPW, _RPW)],
                        out.at[c, pl.ds(s * _RPW, _RPW)])

    return scatter_rows


@functools.cache
def _sc_scatter_fn_r1():
    mesh = plsc.VectorSubcoreMesh(core_axis_name="c", subcore_axis_name="s")

    @functools.partial(
        pl.kernel,
        out_type=jax.ShapeDtypeStruct((_NC, _NPAD, _D), jnp.float32),
        mesh=mesh,
        compiler_params=pltpu.CompilerParams(needs_layout_passes=False),
        scratch_types=[
            pltpu.VMEM((_CH,), jnp.int32),
            pltpu.VMEM((_CH,), jnp.int32),
            pltpu.VMEM((_CH, _D), jnp.float32),
            pltpu.VMEM_SHARED((_NPAD, _D), jnp.float32),
            pltpu.SemaphoreType.DMA,
        ],
    )
    def scatter_rows(table, src, dst, zrows, out, idx_s, idx_d, rows, acc, sem):
        c = lax.axis_index("c")
        s = lax.axis_index("s")
        wid = c * _NS + s
        pltpu.sync_copy(zrows.at[pl.ds(s * _RPW, _RPW)],
                        acc.at[pl.ds(s * _RPW, _RPW)])
        plsc.subcore_barrier()

        def body(j, carry):
            base = pl.multiple_of(wid * _EPW + j * _CH, 8)
            pltpu.sync_copy(src.at[pl.ds(base, _CH)], idx_s)
            pltpu.sync_copy(dst.at[pl.ds(base, _CH)], idx_d)
            pltpu.async_copy(table.at[idx_s], rows, sem).wait()
            pltpu.sync_copy(rows, acc.at[idx_d], add=True)
            return carry

        lax.fori_loop(0, _EPW // _CH, body, 0)
        plsc.subcore_barrier()
        pltpu.sync_copy(acc.at[pl.ds(s * _RPW, _RPW)],
                        out.at[c, pl.ds(s * _RPW, _RPW)])

    return scatter_rows


@functools.cache
def _sc_deg_fn():
    mesh = plsc.VectorSubcoreMesh(core_axis_name="c", subcore_axis_name="s")

    @functools.partial(
        pl.kernel,
        out_type=jax.ShapeDtypeStruct((_NC, _NPAD), jnp.float32),
        mesh=mesh,
        compiler_params=pltpu.CompilerParams(needs_layout_passes=False),
        scratch_types=[
            pltpu.VMEM((_NPAD,), jnp.float32),
            pltpu.VMEM((_DCH,), jnp.int32),
            pltpu.VMEM((_NS, _RPW), jnp.float32),
            pltpu.VMEM((_RPW,), jnp.float32),
            pltpu.VMEM_SHARED((_NS, _NPAD), jnp.float32),
        ],
    )
    def deg_kernel(dst, zdeg, out, hist, idxbuf, buf, res, stage):
        c = lax.axis_index("c")
        s = lax.axis_index("s")
        wid = c * _NS + s
        pltpu.sync_copy(zdeg, hist)
        ones16 = jnp.full((16,), 1.0, jnp.float32)
        for jc in range(_EPW // _DCH):
            base = pl.multiple_of(wid * _EPW + jc * _DCH, 8)
            pltpu.sync_copy(dst.at[pl.ds(base, _DCH)], idxbuf)

            def body(k, carry):
                idx16 = idxbuf[pl.ds(k * 16, 16)]
                plsc.addupdate_scatter(hist, [idx16], ones16)
                return carry

            lax.fori_loop(0, _DCH // 16, body, 0)
        pltpu.sync_copy(hist, stage.at[s])
        plsc.subcore_barrier()
        pltpu.sync_copy(stage.at[:, pl.ds(s * _RPW, _RPW)], buf)

        def rbody(j, carry):
            v = buf[0, pl.ds(j * 16, 16)]
            for t in range(1, _NS):
                v = v + buf[t, pl.ds(j * 16, 16)]
            res[pl.ds(j * 16, 16)] = v
            return carry

        lax.fori_loop(0, _RPW // 16, rbody, 0)
        pltpu.sync_copy(res, out.at[c, pl.ds(s * _RPW, _RPW)])

    return deg_kernel


def _tc_pre(x, dega, degb, W1, b1, W2, b2):
    def body(x_ref, da, db, w1, b1r, w2, b2r, h_ref, hp_ref, dinv_ref):
        deg = da[...] + db[...]
        good = deg > 0
        dinv = jnp.where(good, lax.rsqrt(jnp.where(good, deg, 1.0)), 0.0)
        z = _act(jnp.dot(x_ref[...], w1[...],
                         preferred_element_type=jnp.float32) + b1r[...])
        h = _act(jnp.dot(z, w2[...],
                         preferred_element_type=jnp.float32) + b2r[...])
        h_ref[...] = h
        hp_ref[...] = h * dinv
        dinv_ref[...] = dinv

    rowspec = pl.BlockSpec((_R, _D), lambda i: (i, 0))
    colspec = pl.BlockSpec((_R, 1), lambda i: (i, 0))
    wspec = pl.BlockSpec((_D, _D), lambda i: (0, 0))
    bspec = pl.BlockSpec((1, _D), lambda i: (0, 0))
    return pl.pallas_call(
        body,
        grid=(_NBLK,),
        in_specs=[rowspec, colspec, colspec, wspec, bspec, wspec, bspec],
        out_specs=[rowspec, rowspec, colspec],
        out_shape=[
            jax.ShapeDtypeStruct((_NPAD, _D), jnp.float32),
            jax.ShapeDtypeStruct((_NPAD, _D), jnp.float32),
            jax.ShapeDtypeStruct((_NPAD, 1), jnp.float32),
        ],
    )(x, dega, degb, W1, b1.reshape(1, _D), W2, b2.reshape(1, _D))


def _tc_mid(s1a, s1b, dinv):
    def body(a, b, dv_ref, tx1_ref, u_ref):
        dv = dv_ref[...]
        tx1 = -dv * (a[...] + b[...])
        tx1_ref[...] = tx1
        u_ref[...] = dv * tx1

    rowspec = pl.BlockSpec((_R, _D), lambda i: (i, 0))
    colspec = pl.BlockSpec((_R, 1), lambda i: (i, 0))
    return pl.pallas_call(
        body,
        grid=(_NBLK,),
        in_specs=[rowspec, rowspec, colspec],
        out_specs=[rowspec, rowspec],
        out_shape=[
            jax.ShapeDtypeStruct((_NPAD, _D), jnp.float32),
            jax.ShapeDtypeStruct((_NPAD, _D), jnp.float32),
        ],
    )(s1a, s1b, dinv)


def _tc_post(h, tx1, s2a, s2b, dinv, gp, gpT, Wc0, Wc1, Wc2, bc,
             W3, b3, W4, b4, xLx, W8, b8, W9, b9, W5, b5, W6, b6,
             W7a, W7b, b7):
    DW = _WIDTH * _D

    def body(h_ref, tx1_ref, a_ref, b_ref, dv_ref, gp_ref, gpt_ref,
             wc0, wc1, wc2, bcr, w3, b3r, w4, b4r, xlx, w8, b8r, w9, b9r,
             w5, b5r, w6, b6r, w7a, w7b, b7r, out_ref, hg_acc):
        i = pl.program_id(0)

        @pl.when(i == 0)
        def _():
            hg_acc[...] = jnp.zeros_like(hg_acc)

        tx2 = -2.0 * dv_ref[...] * (a_ref[...] + b_ref[...]) - h_ref[...]
        hf = (jnp.dot(h_ref[...], wc0[...], preferred_element_type=jnp.float32)
              + jnp.dot(tx1_ref[...], wc1[...], preferred_element_type=jnp.float32)
              + jnp.dot(tx2, wc2[...], preferred_element_type=jnp.float32)
              + bcr[...])
        h2 = _act(jnp.dot(hf, w3[...], preferred_element_type=jnp.float32) + b3r[...])
        h2 = _act(jnp.dot(h2, w4[...], preferred_element_type=jnp.float32) + b4r[...])
        t = _act(jnp.dot(xlx[...], w8[...], preferred_element_type=jnp.float32) + b8r[...])
        t = _act(jnp.dot(t, w9[...], preferred_element_type=jnp.float32) + b9r[...])
        oh = (gpt_ref[...] > 0).astype(jnp.float32)
        tg = jnp.dot(oh, t, preferred_element_type=jnp.float32)
        scores = jnp.sum(h2 * tg, axis=1, keepdims=True)
        w = h2 * scores
        hg_acc[...] += jnp.dot(gp_ref[...], w, preferred_element_type=jnp.float32)

        @pl.when(i == _NBLK - 1)
        def _():
            xl = jnp.dot(xlx[...], w5[...], preferred_element_type=jnp.float32) + b5r[...]
            xl = _act(jnp.dot(xl, w6[...], preferred_element_type=jnp.float32) + b6r[...])
            out_ref[...] = (
                jnp.dot(hg_acc[...], w7a[...], preferred_element_type=jnp.float32)
                + jnp.dot(xl, w7b[...], preferred_element_type=jnp.float32)
                + b7r[...])

    rowspec = pl.BlockSpec((_R, _D), lambda i: (i, 0))
    colspec = pl.BlockSpec((_R, 1), lambda i: (i, 0))

    def cspec(shape):
        return pl.BlockSpec(shape, lambda i: tuple(0 for _ in shape))

    return pl.pallas_call(
        body,
        grid=(_NBLK,),
        in_specs=[
            rowspec, rowspec, rowspec, rowspec, colspec,
            pl.BlockSpec((_G, _R), lambda i: (0, i)),
            pl.BlockSpec((_R, _G), lambda i: (i, 0)),
            cspec((_D, DW)), cspec((_D, DW)), cspec((_D, DW)), cspec((1, DW)),
            cspec((DW, _H)), cspec((1, _H)), cspec((_H, _H)), cspec((1, _H)),
            cspec((_G, _D)), cspec((_D, _H)), cspec((1, _H)),
            cspec((_H, _H)), cspec((1, _H)),
            cspec((_D, _H)), cspec((1, _H)), cspec((_H, _H)), cspec((1, _H)),
            cspec((_H, _NCLASS)), cspec((_H, _NCLASS)), cspec((1, _NCLASS)),
        ],
        out_specs=pl.BlockSpec((_G, _NCLASS), lambda i: (0, 0)),
        out_shape=jax.ShapeDtypeStruct((_G, _NCLASS), jnp.float32),
        scratch_shapes=[pltpu.VMEM((_G, _H), jnp.float32)],
    )(h, tx1, s2a, s2b, dinv, gp, gpT,
      Wc0, Wc1, Wc2, bc, W3, b3.reshape(1, _H), W4, b4.reshape(1, _H),
      xLx, W8, b8.reshape(1, _H), W9, b9.reshape(1, _H),
      W5, b5.reshape(1, _H), W6, b6.reshape(1, _H),
      W7a, W7b, b7.reshape(1, _NCLASS))


def kernel(features_list, edge_index, xLx_batch, graph_id, graphpool,
           W1, b1, W2, b2, W3, b3, W4, b4, W5, b5, W6, b6, W7, b7,
           W8, b8, W9, b9, cheb_W, cheb_b):
    del graph_id
    src = edge_index[0]
    dst = edge_index[1]
    npad_s = _SRCROWS * _CH - _E
    npad_e = _EPAD - _E
    src2 = jnp.concatenate([src, jnp.full((npad_s,), _N, jnp.int32)]
                           ).reshape(_SRCROWS, _CH)
    padrows = _N + (jnp.arange(npad_e, dtype=jnp.int32) % (_NPAD - _N))
    dst1 = jnp.concatenate([dst, padrows])
    zrows = jnp.zeros((_NPAD, _D), jnp.float32)
    zdeg = jnp.zeros((_NPAD,), jnp.float32)
    xpad = jnp.pad(features_list, ((0, _NPAD - _N), (0, 0)))
    gp = jnp.pad(graphpool, ((0, 0), (0, _NPAD - _N)))

    degs = _sc_deg_fn()(dst, zdeg).reshape(_NC, _NPAD, 1)
    h, hp, dinv = _tc_pre(xpad, degs[0], degs[1], W1, b1, W2, b2)
    s1 = _sc_scatter_fn_r1()(hp, src, dst, zrows)
    tx1, u = _tc_mid(s1[0], s1[1], dinv)
    s2 = _sc_scatter_fn_r1()(u, src, dst, zrows)

    Wc0 = jnp.transpose(cheb_W[:, 0], (1, 0, 2)).reshape(_D, _WIDTH * _D)
    Wc1 = jnp.transpose(cheb_W[:, 1], (1, 0, 2)).reshape(_D, _WIDTH * _D)
    Wc2 = jnp.transpose(cheb_W[:, 2], (1, 0, 2)).reshape(_D, _WIDTH * _D)
    bc = cheb_b.reshape(1, _WIDTH * _D)
    gpT = gp.T

    return _tc_post(h, tx1, s2[0], s2[1], dinv, gp, gpT,
                    Wc0, Wc1, Wc2, bc, W3, b3, W4, b4,
                    xLx_batch, W8, b8, W9, b9, W5, b5, W6, b6,
                    W7[:_H], W7[_H:], b7)


# speedup vs baseline: 2.4044x; 1.5171x over previous
import functools

import jax
import jax.numpy as jnp
from jax import lax
from jax.experimental import pallas as pl
from jax.experimental.pallas import tpu as pltpu
from jax.experimental.pallas import tpu_sc as plsc

_N = 10000
_E = 320000
_D = 128
_G = 64
_H = 64
_WIDTH = 4
_NCLASS = 2

_NC = 2
_NS = 16
_NW = _NC * _NS
_EPW = _E // _NW
_CH = 80
_NCHUNK = 128
_EPAD = _NW * _NCHUNK * _CH
_NCH0 = 184
_NCH1 = 72
_SRCROWS = _NS * _NCH0 + _NS * _NCH1 + (_NCH0 - _NCH1)
_NPAD = 10240
_RPW = _NPAD // _NS
_DROWS = _NPAD // 16
_DRPW = _DROWS // _NS
_DCH = 2000
_NIB = _DROWS // 128

_R = 2048
_NBLK = _NPAD // _R


def _act(z):
    return jnp.where(z >= 0, z, 0.01 * z)


@functools.cache
def _sc_scatter_fn():
    mesh = plsc.VectorSubcoreMesh(core_axis_name="c", subcore_axis_name="s")

    @functools.partial(
        pl.kernel,
        out_type=jax.ShapeDtypeStruct((_NC, _NPAD, _D), jnp.float32),
        mesh=mesh,
        compiler_params=pltpu.CompilerParams(needs_layout_passes=False),
        scratch_types=[
            pltpu.VMEM((_NCH0, _CH), jnp.int32),
            pltpu.VMEM((_CH,), jnp.int32),
            pltpu.VMEM((_CH,), jnp.int32),
            pltpu.VMEM((_CH, _D), jnp.float32),
            pltpu.VMEM((_CH, _D), jnp.float32),
            pltpu.VMEM_SHARED((_NPAD, _D), jnp.float32),
            pltpu.SemaphoreType.DMA,
            pltpu.SemaphoreType.DMA,
            pltpu.SemaphoreType.DMA,
            pltpu.SemaphoreType.DMA,
        ],
    )
    def scatter_rows(table, src2, src1, dst1, zrows, out,
                     srcbuf, dsti0, dsti1, rows0, rows1, acc,
                     semr0, semr1, semd0, semd1):
        c = lax.axis_index("c")
        s = lax.axis_index("s")
        cb = pl.multiple_of(
            jnp.where(c == 0, s * _NCH0, _NS * _NCH0 + s * _NCH1), 8)
        nch = jnp.where(c == 0, _NCH0, _NCH1)
        ebase = pl.multiple_of(cb * _CH, 8)
        pltpu.sync_copy(src2.at[pl.ds(cb, _NCH0)], srcbuf)
        pltpu.sync_copy(zrows.at[pl.ds(s * _RPW, _RPW)],
                        acc.at[pl.ds(s * _RPW, _RPW)])
        plsc.subcore_barrier()

        @pl.when(c == 0)
        def _():
            pltpu.async_copy(table.at[srcbuf.at[0]], rows0, semr0)
            pltpu.async_copy(dst1.at[pl.ds(ebase, _CH)], dsti0, semd0)
            pltpu.async_copy(table.at[srcbuf.at[1]], rows1, semr1)
            pltpu.async_copy(dst1.at[pl.ds(ebase + _CH, _CH)], dsti1, semd1)

            def body(i, carry):
                j0 = 2 * i
                pltpu.make_async_copy(table.at[srcbuf.at[0]], rows0, semr0).wait()
                pltpu.make_async_copy(dst1.at[pl.ds(0, _CH)], dsti0, semd0).wait()
                pltpu.sync_copy(rows0, acc.at[dsti0], add=True)

                @pl.when(j0 + 2 < _NCH0)
                def _():
                    pltpu.async_copy(table.at[srcbuf.at[j0 + 2]], rows0, semr0)
                    pltpu.async_copy(
                        dst1.at[pl.ds(ebase + (j0 + 2) * _CH, _CH)], dsti0, semd0)

                pltpu.make_async_copy(table.at[srcbuf.at[1]], rows1, semr1).wait()
                pltpu.make_async_copy(dst1.at[pl.ds(0, _CH)], dsti1, semd1).wait()
                pltpu.sync_copy(rows1, acc.at[dsti1], add=True)

                @pl.when(j0 + 3 < _NCH0)
                def _():
                    pltpu.async_copy(table.at[srcbuf.at[j0 + 3]], rows1, semr1)
                    pltpu.async_copy(
                        dst1.at[pl.ds(ebase + (j0 + 3) * _CH, _CH)], dsti1, semd1)

                return carry

            lax.fori_loop(0, 0, body, 0)

        @pl.when(c == 1)
        def _():
            def sbody(j, carry):
                pltpu.sync_copy(src1.at[pl.ds(ebase + j * _CH, _CH)], dsti1)
                pltpu.sync_copy(dst1.at[pl.ds(ebase + j * _CH, _CH)], dsti0)
                pltpu.async_copy(table.at[dsti1], rows0, semr0).wait()
                pltpu.sync_copy(rows0, acc.at[dsti0], add=True)
                return carry

            lax.fori_loop(0, _NCH1, sbody, 0)

        plsc.subcore_barrier()
        pltpu.sync_copy(acc.at[pl.ds(s * _RPW, _RPW)],
                        out.at[c, pl.ds(s * _RPW, _RPW)])

    return scatter_rows


@functools.cache
def _sc_scatter_fn_r1():
    mesh = plsc.VectorSubcoreMesh(core_axis_name="c", subcore_axis_name="s")

    @functools.partial(
        pl.kernel,
        out_type=jax.ShapeDtypeStruct((_NC, _NPAD, _D), jnp.float32),
        mesh=mesh,
        compiler_params=pltpu.CompilerParams(needs_layout_passes=False),
        scratch_types=[
            pltpu.VMEM((_CH,), jnp.int32),
            pltpu.VMEM((_CH,), jnp.int32),
            pltpu.VMEM((_CH,), jnp.int32),
            pltpu.VMEM((_CH,), jnp.int32),
            pltpu.VMEM((_CH, _D), jnp.float32),
            pltpu.VMEM((_CH, _D), jnp.float32),
            pltpu.VMEM_SHARED((_NPAD, _D), jnp.float32),
            pltpu.SemaphoreType.DMA,
            pltpu.SemaphoreType.DMA,
        ],
    )
    def scatter_rows(table, src, dst, zrows, out,
                     idx_sa, idx_da, idx_sb, idx_db, rowsa, rowsb, acc,
                     sema, semb):
        c = lax.axis_index("c")
        s = lax.axis_index("s")
        wid = c * _NS + s
        ebase = wid * _EPW
        pltpu.sync_copy(zrows.at[pl.ds(s * _RPW, _RPW)],
                        acc.at[pl.ds(s * _RPW, _RPW)])
        plsc.subcore_barrier()

        pltpu.sync_copy(src.at[pl.ds(ebase, _CH)], idx_sa)
        pltpu.sync_copy(dst.at[pl.ds(ebase, _CH)], idx_da)
        pltpu.async_copy(table.at[idx_sa], rowsa, sema)

        def body(i, carry):
            j1 = 2 * i + 1
            baseb = pl.multiple_of(ebase + j1 * _CH, 8)
            pltpu.sync_copy(src.at[pl.ds(baseb, _CH)], idx_sb)
            pltpu.sync_copy(dst.at[pl.ds(baseb, _CH)], idx_db)
            pltpu.async_copy(table.at[idx_sb], rowsb, semb)
            pltpu.make_async_copy(table.at[idx_sa], rowsa, sema).wait()
            pltpu.sync_copy(rowsa, acc.at[idx_da], add=True)
            basea = pl.multiple_of(ebase + (j1 + 1) * _CH, 8)
            pltpu.sync_copy(src.at[pl.ds(basea, _CH)], idx_sa)
            pltpu.sync_copy(dst.at[pl.ds(basea, _CH)], idx_da)
            pltpu.async_copy(table.at[idx_sa], rowsa, sema)
            pltpu.make_async_copy(table.at[idx_sb], rowsb, semb).wait()
            pltpu.sync_copy(rowsb, acc.at[idx_db], add=True)
            return carry

        lax.fori_loop(0, (_EPW // _CH) // 2, body, 0)
        pltpu.make_async_copy(table.at[idx_sa], rowsa, sema).wait()
        pltpu.sync_copy(rowsa, acc.at[idx_da], add=True)
        plsc.subcore_barrier()
        pltpu.sync_copy(acc.at[pl.ds(s * _RPW, _RPW)],
                        out.at[c, pl.ds(s * _RPW, _RPW)])

    return scatter_rows


@functools.cache
def _sc_deg_fn():
    mesh = plsc.VectorSubcoreMesh(core_axis_name="c", subcore_axis_name="s")

    @functools.partial(
        pl.kernel,
        out_type=jax.ShapeDtypeStruct((_NC, _NPAD), jnp.float32),
        mesh=mesh,
        compiler_params=pltpu.CompilerParams(needs_layout_passes=False),
        scratch_types=[
            pltpu.VMEM((_NPAD,), jnp.float32),
            pltpu.VMEM((_DCH,), jnp.int32),
            pltpu.VMEM((_NS, _RPW), jnp.float32),
            pltpu.VMEM((_RPW,), jnp.float32),
            pltpu.VMEM_SHARED((_NS, _NPAD), jnp.float32),
        ],
    )
    def deg_kernel(dst, zdeg, out, hist, idxbuf, buf, res, stage):
        c = lax.axis_index("c")
        s = lax.axis_index("s")
        wid = c * _NS + s
        pltpu.sync_copy(zdeg, hist)
        ones16 = jnp.full((16,), 1.0, jnp.float32)
        for jc in range(_EPW // _DCH):
            base = pl.multiple_of(wid * _EPW + jc * _DCH, 8)
            pltpu.sync_copy(dst.at[pl.ds(base, _DCH)], idxbuf)

            def body(k, carry):
                idx16 = idxbuf[pl.ds(k * 16, 16)]
                plsc.addupdate_scatter(hist, [idx16], ones16)
                return carry

            lax.fori_loop(0, _DCH // 16, body, 0)
        pltpu.sync_copy(hist, stage.at[s])
        plsc.subcore_barrier()
        pltpu.sync_copy(stage.at[:, pl.ds(s * _RPW, _RPW)], buf)

        def rbody(j, carry):
            v = buf[0, pl.ds(j * 16, 16)]
            for t in range(1, _NS):
                v = v + buf[t, pl.ds(j * 16, 16)]
            res[pl.ds(j * 16, 16)] = v
            return carry

        lax.fori_loop(0, _RPW // 16, rbody, 0)
        pltpu.sync_copy(res, out.at[c, pl.ds(s * _RPW, _RPW)])

    return deg_kernel


def _tc_pre(x, dega, degb, W1, b1, W2, b2):
    def body(x_ref, da, db, w1, b1r, w2, b2r, h_ref, hp_ref, dinv_ref):
        deg = da[...] + db[...]
        good = deg > 0
        dinv = jnp.where(good, lax.rsqrt(jnp.where(good, deg, 1.0)), 0.0)
        z = _act(jnp.dot(x_ref[...], w1[...],
                         preferred_element_type=jnp.float32) + b1r[...])
        h = _act(jnp.dot(z, w2[...],
                         preferred_element_type=jnp.float32) + b2r[...])
        h_ref[...] = h
        hp_ref[...] = h * dinv
        dinv_ref[...] = dinv

    rowspec = pl.BlockSpec((_R, _D), lambda i: (i, 0))
    colspec = pl.BlockSpec((_R, 1), lambda i: (i, 0))
    wspec = pl.BlockSpec((_D, _D), lambda i: (0, 0))
    bspec = pl.BlockSpec((1, _D), lambda i: (0, 0))
    return pl.pallas_call(
        body,
        grid=(_NBLK,),
        in_specs=[rowspec, colspec, colspec, wspec, bspec, wspec, bspec],
        out_specs=[rowspec, rowspec, colspec],
        out_shape=[
            jax.ShapeDtypeStruct((_NPAD, _D), jnp.float32),
            jax.ShapeDtypeStruct((_NPAD, _D), jnp.float32),
            jax.ShapeDtypeStruct((_NPAD, 1), jnp.float32),
        ],
    )(x, dega, degb, W1, b1.reshape(1, _D), W2, b2.reshape(1, _D))


def _tc_mid(s1a, s1b, dinv):
    def body(a, b, dv_ref, tx1_ref, u_ref):
        dv = dv_ref[...]
        tx1 = -dv * (a[...] + b[...])
        tx1_ref[...] = tx1
        u_ref[...] = dv * tx1

    rowspec = pl.BlockSpec((_R, _D), lambda i: (i, 0))
    colspec = pl.BlockSpec((_R, 1), lambda i: (i, 0))
    return pl.pallas_call(
        body,
        grid=(_NBLK,),
        in_specs=[rowspec, rowspec, colspec],
        out_specs=[rowspec, rowspec],
        out_shape=[
            jax.ShapeDtypeStruct((_NPAD, _D), jnp.float32),
            jax.ShapeDtypeStruct((_NPAD, _D), jnp.float32),
        ],
    )(s1a, s1b, dinv)


def _tc_post(h, tx1, s2a, s2b, dinv, gp, gpT, Wc0, Wc1, Wc2, bc,
             W3, b3, W4, b4, xLx, W8, b8, W9, b9, W5, b5, W6, b6,
             W7a, W7b, b7):
    DW = _WIDTH * _D

    def body(h_ref, tx1_ref, a_ref, b_ref, dv_ref, gp_ref, gpt_ref,
             wc0, wc1, wc2, bcr, w3, b3r, w4, b4r, xlx, w8, b8r, w9, b9r,
             w5, b5r, w6, b6r, w7a, w7b, b7r, out_ref, hg_acc):
        i = pl.program_id(0)

        @pl.when(i == 0)
        def _():
            hg_acc[...] = jnp.zeros_like(hg_acc)

        tx2 = -2.0 * dv_ref[...] * (a_ref[...] + b_ref[...]) - h_ref[...]
        hf = (jnp.dot(h_ref[...], wc0[...], preferred_element_type=jnp.float32)
              + jnp.dot(tx1_ref[...], wc1[...], preferred_element_type=jnp.float32)
              + jnp.dot(tx2, wc2[...], preferred_element_type=jnp.float32)
              + bcr[...])
        h2 = _act(jnp.dot(hf, w3[...], preferred_element_type=jnp.float32) + b3r[...])
        h2 = _act(jnp.dot(h2, w4[...], preferred_element_type=jnp.float32) + b4r[...])
        t = _act(jnp.dot(xlx[...], w8[...], preferred_element_type=jnp.float32) + b8r[...])
        t = _act(jnp.dot(t, w9[...], preferred_element_type=jnp.float32) + b9r[...])
        oh = (gpt_ref[...] > 0).astype(jnp.float32)
        tg = jnp.dot(oh, t, preferred_element_type=jnp.float32)
        scores = jnp.sum(h2 * tg, axis=1, keepdims=True)
        w = h2 * scores
        hg_acc[...] += jnp.dot(gp_ref[...], w, preferred_element_type=jnp.float32)

        @pl.when(i == _NBLK - 1)
        def _():
            xl = jnp.dot(xlx[...], w5[...], preferred_element_type=jnp.float32) + b5r[...]
            xl = _act(jnp.dot(xl, w6[...], preferred_element_type=jnp.float32) + b6r[...])
            out_ref[...] = (
                jnp.dot(hg_acc[...], w7a[...], preferred_element_type=jnp.float32)
                + jnp.dot(xl, w7b[...], preferred_element_type=jnp.float32)
                + b7r[...])

    rowspec = pl.BlockSpec((_R, _D), lambda i: (i, 0))
    colspec = pl.BlockSpec((_R, 1), lambda i: (i, 0))

    def cspec(shape):
        return pl.BlockSpec(shape, lambda i: tuple(0 for _ in shape))

    return pl.pallas_call(
        body,
        grid=(_NBLK,),
        in_specs=[
            rowspec, rowspec, rowspec, rowspec, colspec,
            pl.BlockSpec((_G, _R), lambda i: (0, i)),
            pl.BlockSpec((_R, _G), lambda i: (i, 0)),
            cspec((_D, DW)), cspec((_D, DW)), cspec((_D, DW)), cspec((1, DW)),
            cspec((DW, _H)), cspec((1, _H)), cspec((_H, _H)), cspec((1, _H)),
            cspec((_G, _D)), cspec((_D, _H)), cspec((1, _H)),
            cspec((_H, _H)), cspec((1, _H)),
            cspec((_D, _H)), cspec((1, _H)), cspec((_H, _H)), cspec((1, _H)),
            cspec((_H, _NCLASS)), cspec((_H, _NCLASS)), cspec((1, _NCLASS)),
        ],
        out_specs=pl.BlockSpec((_G, _NCLASS), lambda i: (0, 0)),
        out_shape=jax.ShapeDtypeStruct((_G, _NCLASS), jnp.float32),
        scratch_shapes=[pltpu.VMEM((_G, _H), jnp.float32)],
    )(h, tx1, s2a, s2b, dinv, gp, gpT,
      Wc0, Wc1, Wc2, bc, W3, b3.reshape(1, _H), W4, b4.reshape(1, _H),
      xLx, W8, b8.reshape(1, _H), W9, b9.reshape(1, _H),
      W5, b5.reshape(1, _H), W6, b6.reshape(1, _H),
      W7a, W7b, b7.reshape(1, _NCLASS))


def kernel(features_list, edge_index, xLx_batch, graph_id, graphpool,
           W1, b1, W2, b2, W3, b3, W4, b4, W5, b5, W6, b6, W7, b7,
           W8, b8, W9, b9, cheb_W, cheb_b):
    del graph_id
    src = edge_index[0]
    dst = edge_index[1]
    npad_s = _SRCROWS * _CH - _E
    npad_e = _EPAD - _E
    src2 = jnp.concatenate([src, jnp.full((npad_s,), _N, jnp.int32)]
                           ).reshape(_SRCROWS, _CH)
    padrows = _N + (jnp.arange(npad_e, dtype=jnp.int32) % (_NPAD - _N))
    dst1 = jnp.concatenate([dst, padrows])
    zrows = jnp.zeros((_NPAD, _D), jnp.float32)
    zdeg = jnp.zeros((_NPAD,), jnp.float32)
    xpad = jnp.pad(features_list, ((0, _NPAD - _N), (0, 0)))
    gp = jnp.pad(graphpool, ((0, 0), (0, _NPAD - _N)))

    degs = _sc_deg_fn()(dst, zdeg).reshape(_NC, _NPAD, 1)
    h, hp, dinv = _tc_pre(xpad, degs[0], degs[1], W1, b1, W2, b2)
    s1 = _sc_scatter_fn_r1()(hp, src, dst, zrows)
    tx1, u = _tc_mid(s1[0], s1[1], dinv)
    s2 = _sc_scatter_fn_r1()(u, src, dst, zrows)

    Wc0 = jnp.transpose(cheb_W[:, 0], (1, 0, 2)).reshape(_D, _WIDTH * _D)
    Wc1 = jnp.transpose(cheb_W[:, 1], (1, 0, 2)).reshape(_D, _WIDTH * _D)
    Wc2 = jnp.transpose(cheb_W[:, 2], (1, 0, 2)).reshape(_D, _WIDTH * _D)
    bc = cheb_b.reshape(1, _WIDTH * _D)
    gpT = gp.T

    return _tc_post(h, tx1, s2[0], s2[1], dinv, gp, gpT,
                    Wc0, Wc1, Wc2, bc, W3, b3, W4, b4,
                    xLx_batch, W8, b8, W9, b9, W5, b5, W6, b6,
                    W7[:_H], W7[_H:], b7)
